# baseline (speedup 1.0000x reference)
import jax
import jax.numpy as jnp
from jax import lax
from jax.experimental import pallas as pl
from jax.experimental.pallas import tpu as pltpu
from jax.experimental.pallas import tpu_sc as plsc

B = 16384
K = 50
DIM = 16
NC = 2
NS = 16
NW = NC * NS
RPW = B // NW
C = 64
NCHUNK = RPW // C
NG = C // 16
NE = 1000000
EBLK = 8192
NBLK = (NE + EBLK - 1) // EBLK
ZROWS = NBLK * (EBLK // 8)
ZVIEW = ZROWS * 8

_PARAMS = dict(
    compiler_params=pltpu.CompilerParams(
        needs_layout_passes=False, use_tc_tiling_on_sc=False),
)


def _tree_sum(xs):
    xs = list(xs)
    while len(xs) > 1:
        xs = [xs[i] + xs[i + 1] for i in range(0, len(xs) - 1, 2)] + \
             ([xs[-1]] if len(xs) % 2 else [])
    return xs[0]


def _rsqrt(q):
    i = plsc.bitcast(q, jnp.int32)
    y = plsc.bitcast(jnp.int32(0x5F3759DF) - (i >> 1), jnp.float32)
    for _ in range(3):
        y = y * (1.5 - 0.5 * q * y * y)
    return y


def _perm_ids(e):
    e = e.astype(jnp.int32)
    return ((e >> 10) << 10) | ((e & 127) << 3) | ((e >> 7) & 7)


def _relayout_body(src, dst):
    x = src[...]
    for s in range(EBLK // 1024):
        for j in range(8):
            dst[128 * s:128 * (s + 1), 16 * j:16 * (j + 1)] = \
                x[:, 1024 * s + 128 * j:1024 * s + 128 * (j + 1)].T


def _stage_wa(w_hbm, a_hbm, w_v, a_v):
    pltpu.sync_copy(w_hbm, w_v)
    pltpu.sync_copy(a_hbm, a_v)
    arow = a_v[0, :]
    wrows = [w_v[i, :] for i in range(8)]
    return arow, wrows


def _body_a(nb_hbm, tid_hbm, et_hbm, w_hbm, a_hbm, y_hbm,
            idx_v, nbr_v, tid_v, trow_v, w_v, a_v, y_v, acc_v, sem):
    cid = lax.axis_index("c")
    sid = lax.axis_index("s")
    wid = sid * NC + cid

    arow, wrows = _stage_wa(w_hbm, a_hbm, w_v, a_v)
    v1 = jnp.zeros((16,), jnp.float32)
    v2 = jnp.zeros((16,), jnp.float32)
    for i in range(8):
        v1 = v1 + wrows[i] * arow[i]
        v2 = v2 + wrows[i] * arow[8 + i]
    v1s = [v1[d] for d in range(16)]
    v2s = [v2[d] for d in range(16)]

    iota = lax.iota(jnp.int32, 16)
    cds = [jnp.full((16,), d, jnp.int32) for d in range(16)]

    pltpu.sync_copy(nb_hbm.at[wid], idx_v)
    pltpu.sync_copy(tid_hbm.at[wid], tid_v)
    pltpu.async_copy(et_hbm.at[tid_v], trow_v, sem).wait()

    def chunk_body(c, carry):
        cp = pltpu.async_copy(et_hbm.at[idx_v.at[pl.ds(c * (C * K), C * K)]],
                              nbr_v, sem)
        cp.wait()

        def group_body(g, carry2):
            row16 = c * C + g * 16 + iota

            tcols = [plsc.load_gather(trow_v, [row16, cds[d]])
                     for d in range(16)]
            tq = _tree_sum([c2 * c2 for c2 in tcols])
            tp = _tree_sum([tcols[d] * v1s[d] for d in range(16)])
            st = tp * jnp.minimum(1.0, _rsqrt(tq))

            row50 = (g * 16 + iota) * K
            zero = jnp.zeros((16,), jnp.float32)

            for d in range(16):
                acc_v[pl.ds(d * 16, 16)] = zero

            @plsc.parallel_loop(0, K, unroll=2, carry=zero)
            def k_loop(k, ssum):
                r = row50 + k
                cols = [plsc.load_gather(nbr_v, [r, cds[d]])
                        for d in range(16)]
                q = _tree_sum([c2 * c2 for c2 in cols])
                p = _tree_sum([cols[d] * v2s[d] for d in range(16)])
                scl = jnp.minimum(1.0, _rsqrt(q))
                e = st + p * scl
                e = jnp.maximum(e, e * 0.2)
                w = jnp.exp(e)
                t = w * scl
                for d in range(16):
                    plsc.addupdate(acc_v.at[pl.ds(d * 16, 16)], t * cols[d])
                return ssum + w

            inv = 1.0 / k_loop
            for d in range(16):
                plsc.store_scatter(y_v, [row16, cds[d]],
                                   acc_v[pl.ds(d * 16, 16)] * inv)
            return 0

        lax.fori_loop(0, NG, group_body, 0)
        return 0

    lax.fori_loop(0, NCHUNK, chunk_body, 0)
    pltpu.sync_copy(y_v, y_hbm.at[pl.ds(wid * RPW, RPW)])


def _body_b(y_hbm, uid_hbm, ut_hbm, w_hbm, a_hbm, out_hbm,
            uid_v, urow_v, y_v, w_v, a_v, out_v, sem):
    cid = lax.axis_index("c")
    sid = lax.axis_index("s")
    wid = sid * NC + cid

    _, wrows = _stage_wa(w_hbm, a_hbm, w_v, a_v)
    iota = lax.iota(jnp.int32, 16)
    cds = [jnp.full((16,), d, jnp.int32) for d in range(16)]

    pltpu.sync_copy(uid_hbm.at[wid], uid_v)
    pltpu.sync_copy(y_hbm.at[pl.ds(wid * RPW, RPW)], y_v)
    pltpu.async_copy(ut_hbm.at[uid_v], urow_v, sem).wait()

    def group_body(g, carry):
        row16 = g * 16 + iota
        ucols = [plsc.load_gather(urow_v, [row16, cds[d]])
                 for d in range(16)]
        uq = _tree_sum([c2 * c2 for c2 in ucols])
        us = jnp.minimum(1.0, _rsqrt(uq))
        gs = [ucols[i] + ucols[8 + i] for i in range(8)]
        uv = jnp.zeros((16,), jnp.float32)
        for j in range(16):
            wv = gs[0] * wrows[0][j]
            for i in range(1, 8):
                wv = wv + gs[i] * wrows[i][j]
            yj = plsc.load_gather(y_v, [row16, cds[j]])
            uv = uv + yj * wv
        uv = uv * us
        logit = 1.0 / (1.0 + jnp.exp(-uv))
        out_v[pl.ds(g * 16, 16)] = logit
        return 0

    lax.fori_loop(0, RPW // 16, group_body, 0)
    pltpu.sync_copy(out_v, out_hbm.at[pl.ds(wid * RPW, RPW)])


def kernel(u, target_ids, neighbor_ids, entity_table, user_table, W, a):
    relayout = pl.pallas_call(
        _relayout_body,
        grid=(NBLK,),
        in_specs=[pl.BlockSpec((16, EBLK), lambda i: (0, i))],
        out_specs=pl.BlockSpec((EBLK // 8, 128), lambda i: (i, 0)),
        out_shape=jax.ShapeDtypeStruct((ZROWS, 128), jnp.float32),
    )
    et16 = relayout(entity_table.T).reshape(ZVIEW, DIM)
    ut16 = relayout(user_table.T).reshape(ZVIEW, DIM)

    nb2d = _perm_ids(neighbor_ids).reshape(NW, RPW * K)
    tids = _perm_ids(target_ids).reshape(NW, RPW)
    uids = _perm_ids(u).reshape(NW, RPW)
    mesh = plsc.VectorSubcoreMesh(core_axis_name="c", subcore_axis_name="s")
    fn_a = pl.kernel(
        _body_a,
        out_type=jax.ShapeDtypeStruct((B, DIM), jnp.float32),
        mesh=mesh,
        scratch_types=[
            pltpu.VMEM((RPW * K,), jnp.int32),
            pltpu.VMEM((C * K, DIM), jnp.float32),
            pltpu.VMEM((RPW,), jnp.int32),
            pltpu.VMEM((RPW, DIM), jnp.float32),
            pltpu.VMEM((8, DIM), jnp.float32),
            pltpu.VMEM((1, DIM), jnp.float32),
            pltpu.VMEM((RPW, DIM), jnp.float32),
            pltpu.VMEM((16 * DIM,), jnp.float32),
            pltpu.SemaphoreType.DMA,
        ],
        **_PARAMS,
    )
    y = fn_a(nb2d, tids, et16, W, a)
    fn_b = pl.kernel(
        _body_b,
        out_type=jax.ShapeDtypeStruct((B,), jnp.float32),
        mesh=mesh,
        scratch_types=[
            pltpu.VMEM((RPW,), jnp.int32),
            pltpu.VMEM((RPW, DIM), jnp.float32),
            pltpu.VMEM((RPW, DIM), jnp.float32),
            pltpu.VMEM((8, DIM), jnp.float32),
            pltpu.VMEM((1, DIM), jnp.float32),
            pltpu.VMEM((RPW,), jnp.float32),
            pltpu.SemaphoreType.DMA,
        ],
        **_PARAMS,
    )
    return fn_b(y, uids, ut16, W, a)


# speedup vs baseline: 1.0244x; 1.0244x over previous
import jax
import jax.numpy as jnp
from jax import lax
from jax.experimental import pallas as pl
from jax.experimental.pallas import tpu as pltpu
from jax.experimental.pallas import tpu_sc as plsc

B = 16384
K = 50
DIM = 16
NC = 2
NS = 16
NW = NC * NS
RPW = B // NW
C = 64
NCHUNK = RPW // C
NG = C // 16
NE = 1000000
EBLK = 8192
NBLK = (NE + EBLK - 1) // EBLK
ZROWS = NBLK * (EBLK // 8)
ZVIEW = ZROWS * 8

_PARAMS = dict(
    compiler_params=pltpu.CompilerParams(
        needs_layout_passes=False, use_tc_tiling_on_sc=False),
)


def _tree_sum(xs):
    xs = list(xs)
    while len(xs) > 1:
        xs = [xs[i] + xs[i + 1] for i in range(0, len(xs) - 1, 2)] + \
             ([xs[-1]] if len(xs) % 2 else [])
    return xs[0]


def _rsqrt(q):
    i = plsc.bitcast(q, jnp.int32)
    y = plsc.bitcast(jnp.int32(0x5F3759DF) - (i >> 1), jnp.float32)
    for _ in range(3):
        y = y * (1.5 - 0.5 * q * y * y)
    return y


def _perm_ids(e):
    e = e.astype(jnp.int32)
    return ((e >> 10) << 10) | ((e & 127) << 3) | ((e >> 7) & 7)


def _relayout_body(src, dst):
    x = src[...]
    for s in range(EBLK // 1024):
        for j in range(8):
            dst[128 * s:128 * (s + 1), 16 * j:16 * (j + 1)] = \
                x[:, 1024 * s + 128 * j:1024 * s + 128 * (j + 1)].T


def _stage_wa(w_hbm, a_hbm, w_v, a_v):
    pltpu.sync_copy(w_hbm, w_v)
    pltpu.sync_copy(a_hbm, a_v)
    arow = a_v[0, :]
    wrows = [w_v[i, :] for i in range(8)]
    return arow, wrows


def _body_a(nb_hbm, tid_hbm, et_hbm, w_hbm, a_hbm, y_hbm,
            idx_v, nbr_v, tid_v, trow_v, w_v, a_v, y_v, sem):
    cid = lax.axis_index("c")
    sid = lax.axis_index("s")
    wid = sid * NC + cid

    arow, wrows = _stage_wa(w_hbm, a_hbm, w_v, a_v)
    v1 = jnp.zeros((16,), jnp.float32)
    v2 = jnp.zeros((16,), jnp.float32)
    for i in range(8):
        v1 = v1 + wrows[i] * arow[i]
        v2 = v2 + wrows[i] * arow[8 + i]
    v1s = [v1[d] for d in range(16)]
    v2s = [v2[d] for d in range(16)]

    iota = lax.iota(jnp.int32, 16)
    cds = [jnp.full((16,), d, jnp.int32) for d in range(16)]

    pltpu.sync_copy(nb_hbm.at[wid], idx_v)
    pltpu.sync_copy(tid_hbm.at[wid], tid_v)
    pltpu.async_copy(et_hbm.at[tid_v], trow_v, sem).wait()

    def chunk_body(c, carry):
        cp = pltpu.async_copy(et_hbm.at[idx_v.at[pl.ds(c * (C * K), C * K)]],
                              nbr_v, sem)
        cp.wait()

        def group_body(g, carry2):
            row16 = c * C + g * 16 + iota

            tcols = [plsc.load_gather(trow_v, [row16, cds[d]])
                     for d in range(16)]
            tq = _tree_sum([c2 * c2 for c2 in tcols])
            tp = _tree_sum([tcols[d] * v1s[d] for d in range(16)])
            st = tp * jnp.minimum(1.0, _rsqrt(tq))

            row50 = (g * 16 + iota) * K
            zero = jnp.zeros((16,), jnp.float32)

            @plsc.parallel_loop(0, K, unroll=2, carry=(zero, (zero,) * 16))
            def k_loop(k, kc):
                ssum, acc = kc
                r = row50 + k
                cols = [plsc.load_gather(nbr_v, [r, cds[d]])
                        for d in range(16)]
                q = _tree_sum([c2 * c2 for c2 in cols])
                p = _tree_sum([cols[d] * v2s[d] for d in range(16)])
                scl = jnp.minimum(1.0, _rsqrt(q))
                e = st + p * scl
                e = jnp.maximum(e, e * 0.2)
                w = jnp.exp(e)
                t = w * scl
                acc = tuple(acc[d] + t * cols[d] for d in range(16))
                return (ssum + w, acc)

            ssum, acc = k_loop

            inv = 1.0 / ssum
            for d in range(16):
                plsc.store_scatter(y_v, [row16, cds[d]], acc[d] * inv)
            return 0

        lax.fori_loop(0, NG, group_body, 0)
        return 0

    lax.fori_loop(0, NCHUNK, chunk_body, 0)
    pltpu.sync_copy(y_v, y_hbm.at[pl.ds(wid * RPW, RPW)])


def _body_b(y_hbm, uid_hbm, ut_hbm, w_hbm, a_hbm, out_hbm,
            uid_v, urow_v, y_v, w_v, a_v, out_v, sem):
    cid = lax.axis_index("c")
    sid = lax.axis_index("s")
    wid = sid * NC + cid

    _, wrows = _stage_wa(w_hbm, a_hbm, w_v, a_v)
    iota = lax.iota(jnp.int32, 16)
    cds = [jnp.full((16,), d, jnp.int32) for d in range(16)]

    pltpu.sync_copy(uid_hbm.at[wid], uid_v)
    pltpu.sync_copy(y_hbm.at[pl.ds(wid * RPW, RPW)], y_v)
    pltpu.async_copy(ut_hbm.at[uid_v], urow_v, sem).wait()

    def group_body(g, carry):
        row16 = g * 16 + iota
        ucols = [plsc.load_gather(urow_v, [row16, cds[d]])
                 for d in range(16)]
        uq = _tree_sum([c2 * c2 for c2 in ucols])
        us = jnp.minimum(1.0, _rsqrt(uq))
        gs = [ucols[i] + ucols[8 + i] for i in range(8)]
        uv = jnp.zeros((16,), jnp.float32)
        for j in range(16):
            wv = gs[0] * wrows[0][j]
            for i in range(1, 8):
                wv = wv + gs[i] * wrows[i][j]
            yj = plsc.load_gather(y_v, [row16, cds[j]])
            uv = uv + yj * wv
        uv = uv * us
        logit = 1.0 / (1.0 + jnp.exp(-uv))
        out_v[pl.ds(g * 16, 16)] = logit
        return 0

    lax.fori_loop(0, RPW // 16, group_body, 0)
    pltpu.sync_copy(out_v, out_hbm.at[pl.ds(wid * RPW, RPW)])


def kernel(u, target_ids, neighbor_ids, entity_table, user_table, W, a):
    relayout = pl.pallas_call(
        _relayout_body,
        grid=(NBLK,),
        in_specs=[pl.BlockSpec((16, EBLK), lambda i: (0, i))],
        out_specs=pl.BlockSpec((EBLK // 8, 128), lambda i: (i, 0)),
        out_shape=jax.ShapeDtypeStruct((ZROWS, 128), jnp.float32),
    )
    et16 = relayout(entity_table.T).reshape(ZVIEW, DIM)
    ut16 = relayout(user_table.T).reshape(ZVIEW, DIM)

    nb2d = _perm_ids(neighbor_ids).reshape(NW, RPW * K)
    tids = _perm_ids(target_ids).reshape(NW, RPW)
    uids = _perm_ids(u).reshape(NW, RPW)
    mesh = plsc.VectorSubcoreMesh(core_axis_name="c", subcore_axis_name="s")
    fn_a = pl.kernel(
        _body_a,
        out_type=jax.ShapeDtypeStruct((B, DIM), jnp.float32),
        mesh=mesh,
        scratch_types=[
            pltpu.VMEM((RPW * K,), jnp.int32),
            pltpu.VMEM((C * K, DIM), jnp.float32),
            pltpu.VMEM((RPW,), jnp.int32),
            pltpu.VMEM((RPW, DIM), jnp.float32),
            pltpu.VMEM((8, DIM), jnp.float32),
            pltpu.VMEM((1, DIM), jnp.float32),
            pltpu.VMEM((RPW, DIM), jnp.float32),
            pltpu.SemaphoreType.DMA,
        ],
        **_PARAMS,
    )
    y = fn_a(nb2d, tids, et16, W, a)
    fn_b = pl.kernel(
        _body_b,
        out_type=jax.ShapeDtypeStruct((B,), jnp.float32),
        mesh=mesh,
        scratch_types=[
            pltpu.VMEM((RPW,), jnp.int32),
            pltpu.VMEM((RPW, DIM), jnp.float32),
            pltpu.VMEM((RPW, DIM), jnp.float32),
            pltpu.VMEM((8, DIM), jnp.float32),
            pltpu.VMEM((1, DIM), jnp.float32),
            pltpu.VMEM((RPW,), jnp.float32),
            pltpu.SemaphoreType.DMA,
        ],
        **_PARAMS,
    )
    return fn_b(y, uids, ut16, W, a)


# speedup vs baseline: 1.0866x; 1.0607x over previous
import jax
import jax.numpy as jnp
from jax import lax
from jax.experimental import pallas as pl
from jax.experimental.pallas import tpu as pltpu
from jax.experimental.pallas import tpu_sc as plsc

B = 16384
K = 50
DIM = 16
NC = 2
NS = 16
NW = NC * NS
RPW = B // NW
C = 64
NCHUNK = RPW // C
NG = C // 16
NE = 1000000
EBLK = 8192
NBLK = (NE + EBLK - 1) // EBLK
ZROWS = NBLK * (EBLK // 8)
ZVIEW = ZROWS * 8

_PARAMS = dict(
    compiler_params=pltpu.CompilerParams(
        needs_layout_passes=False, use_tc_tiling_on_sc=False),
)


def _tree_sum(xs):
    xs = list(xs)
    while len(xs) > 1:
        xs = [xs[i] + xs[i + 1] for i in range(0, len(xs) - 1, 2)] + \
             ([xs[-1]] if len(xs) % 2 else [])
    return xs[0]


def _rsqrt(q):
    i = plsc.bitcast(q, jnp.int32)
    y = plsc.bitcast(jnp.int32(0x5F3759DF) - (i >> 1), jnp.float32)
    for _ in range(3):
        y = y * (1.5 - 0.5 * q * y * y)
    return y


def _perm_ids(e):
    e = e.astype(jnp.int32)
    return ((e >> 10) << 10) | ((e & 127) << 3) | ((e >> 7) & 7)


def _relayout_body(src, dst):
    x = src[...]
    for s in range(EBLK // 1024):
        for j in range(8):
            dst[128 * s:128 * (s + 1), 16 * j:16 * (j + 1)] = \
                x[:, 1024 * s + 128 * j:1024 * s + 128 * (j + 1)].T


def _stage_wa(w_hbm, a_hbm, w_v, a_v):
    pltpu.sync_copy(w_hbm, w_v)
    pltpu.sync_copy(a_hbm, a_v)
    arow = a_v[0, :]
    wrows = [w_v[i, :] for i in range(8)]
    return arow, wrows


def _body_a(nb_hbm, tid_hbm, et_hbm, w_hbm, a_hbm, y_hbm,
            idx_v, nbr_v, tid_v, trow_v, w_v, a_v, y_v, sem):
    cid = lax.axis_index("c")
    sid = lax.axis_index("s")
    wid = sid * NC + cid

    arow, wrows = _stage_wa(w_hbm, a_hbm, w_v, a_v)
    v1 = jnp.zeros((16,), jnp.float32)
    v2 = jnp.zeros((16,), jnp.float32)
    for i in range(8):
        v1 = v1 + wrows[i] * arow[i]
        v2 = v2 + wrows[i] * arow[8 + i]
    v1s = [v1[d] for d in range(16)]
    v2s = [v2[d] for d in range(16)]

    iota = lax.iota(jnp.int32, 16)
    cds = [jnp.full((16,), d, jnp.int32) for d in range(16)]

    pltpu.sync_copy(nb_hbm.at[wid], idx_v)
    pltpu.sync_copy(tid_hbm.at[wid], tid_v)
    pltpu.async_copy(et_hbm.at[tid_v], trow_v, sem).wait()

    def chunk_body(c, carry):
        cp = pltpu.async_copy(et_hbm.at[idx_v.at[pl.ds(c * (C * K), C * K)]],
                              nbr_v, sem)
        cp.wait()

        def group_body(g, carry2):
            row16 = c * C + g * 16 + iota

            tcols = [plsc.load_gather(trow_v, [row16, cds[d]])
                     for d in range(16)]
            tq = _tree_sum([c2 * c2 for c2 in tcols])
            tp = _tree_sum([tcols[d] * v1s[d] for d in range(16)])
            st = tp * jnp.minimum(1.0, _rsqrt(tq))

            row50 = (g * 16 + iota) * K
            zero = jnp.zeros((16,), jnp.float32)

            @plsc.parallel_loop(0, K, unroll=2, carry=(zero, (zero,) * 16))
            def k_loop(k, kc):
                ssum, acc = kc
                r = row50 + k
                cols = [plsc.load_gather(nbr_v, [r, cds[d]])
                        for d in range(16)]
                q = _tree_sum([c2 * c2 for c2 in cols])
                p = _tree_sum([cols[d] * v2s[d] for d in range(16)])
                scl = jnp.minimum(1.0, _rsqrt(q))
                e = st + p * scl
                e = jnp.where(e >= 0.0, e, e * 0.2)
                w = jnp.exp(e)
                t = w * scl
                acc = tuple(acc[d] + t * cols[d] for d in range(16))
                return (ssum + w, acc)

            ssum, acc = k_loop

            inv = 1.0 / ssum
            for d in range(16):
                plsc.store_scatter(y_v, [row16, cds[d]], acc[d] * inv)
            return 0

        lax.fori_loop(0, NG, group_body, 0)
        return 0

    lax.fori_loop(0, NCHUNK, chunk_body, 0)
    pltpu.sync_copy(y_v, y_hbm.at[pl.ds(wid * RPW, RPW)])


def _body_b(y_hbm, uid_hbm, ut_hbm, w_hbm, a_hbm, out_hbm,
            uid_v, urow_v, y_v, w_v, a_v, out_v, sem):
    cid = lax.axis_index("c")
    sid = lax.axis_index("s")
    wid = sid * NC + cid

    _, wrows = _stage_wa(w_hbm, a_hbm, w_v, a_v)
    iota = lax.iota(jnp.int32, 16)
    cds = [jnp.full((16,), d, jnp.int32) for d in range(16)]

    pltpu.sync_copy(uid_hbm.at[wid], uid_v)
    pltpu.sync_copy(y_hbm.at[pl.ds(wid * RPW, RPW)], y_v)
    pltpu.async_copy(ut_hbm.at[uid_v], urow_v, sem).wait()

    def group_body(g, carry):
        row16 = g * 16 + iota
        ucols = [plsc.load_gather(urow_v, [row16, cds[d]])
                 for d in range(16)]
        uq = _tree_sum([c2 * c2 for c2 in ucols])
        us = jnp.minimum(1.0, _rsqrt(uq))
        gs = [ucols[i] + ucols[8 + i] for i in range(8)]
        uv = jnp.zeros((16,), jnp.float32)
        for j in range(16):
            wv = gs[0] * wrows[0][j]
            for i in range(1, 8):
                wv = wv + gs[i] * wrows[i][j]
            yj = plsc.load_gather(y_v, [row16, cds[j]])
            uv = uv + yj * wv
        uv = uv * us
        logit = 1.0 / (1.0 + jnp.exp(-uv))
        out_v[pl.ds(g * 16, 16)] = logit
        return 0

    lax.fori_loop(0, RPW // 16, group_body, 0)
    pltpu.sync_copy(out_v, out_hbm.at[pl.ds(wid * RPW, RPW)])


def kernel(u, target_ids, neighbor_ids, entity_table, user_table, W, a):
    relayout = pl.pallas_call(
        _relayout_body,
        grid=(NBLK,),
        in_specs=[pl.BlockSpec((16, EBLK), lambda i: (0, i))],
        out_specs=pl.BlockSpec((EBLK // 8, 128), lambda i: (i, 0)),
        out_shape=jax.ShapeDtypeStruct((ZROWS, 128), jnp.float32),
    )
    et16 = relayout(entity_table.T).reshape(ZVIEW, DIM)
    ut16 = relayout(user_table.T).reshape(ZVIEW, DIM)

    nb2d = _perm_ids(neighbor_ids).reshape(NW, RPW * K)
    tids = _perm_ids(target_ids).reshape(NW, RPW)
    uids = _perm_ids(u).reshape(NW, RPW)
    mesh = plsc.VectorSubcoreMesh(core_axis_name="c", subcore_axis_name="s")
    fn_a = pl.kernel(
        _body_a,
        out_type=jax.ShapeDtypeStruct((B, DIM), jnp.float32),
        mesh=mesh,
        scratch_types=[
            pltpu.VMEM((RPW * K,), jnp.int32),
            pltpu.VMEM((C * K, DIM), jnp.float32),
            pltpu.VMEM((RPW,), jnp.int32),
            pltpu.VMEM((RPW, DIM), jnp.float32),
            pltpu.VMEM((8, DIM), jnp.float32),
            pltpu.VMEM((1, DIM), jnp.float32),
            pltpu.VMEM((RPW, DIM), jnp.float32),
            pltpu.SemaphoreType.DMA,
        ],
        **_PARAMS,
    )
    y = fn_a(nb2d, tids, et16, W, a)
    fn_b = pl.kernel(
        _body_b,
        out_type=jax.ShapeDtypeStruct((B,), jnp.float32),
        mesh=mesh,
        scratch_types=[
            pltpu.VMEM((RPW,), jnp.int32),
            pltpu.VMEM((RPW, DIM), jnp.float32),
            pltpu.VMEM((RPW, DIM), jnp.float32),
            pltpu.VMEM((8, DIM), jnp.float32),
            pltpu.VMEM((1, DIM), jnp.float32),
            pltpu.VMEM((RPW,), jnp.float32),
            pltpu.SemaphoreType.DMA,
        ],
        **_PARAMS,
    )
    return fn_b(y, uids, ut16, W, a)


# speedup vs baseline: 1.1401x; 1.0493x over previous
import jax
import jax.numpy as jnp
from jax import lax
from jax.experimental import pallas as pl
from jax.experimental.pallas import tpu as pltpu
from jax.experimental.pallas import tpu_sc as plsc

B = 16384
K = 50
DIM = 16
NC = 2
NS = 16
NW = NC * NS
RPW = B // NW
C = 32
NCHUNK = RPW // C
NG = C // 16
NE = 1000000
EBLK = 8192
NBLK = (NE + EBLK - 1) // EBLK
ZROWS = NBLK * (EBLK // 8)
ZVIEW = ZROWS * 8

_PARAMS = dict(
    compiler_params=pltpu.CompilerParams(
        needs_layout_passes=False, use_tc_tiling_on_sc=False),
)


def _tree_sum(xs):
    xs = list(xs)
    while len(xs) > 1:
        xs = [xs[i] + xs[i + 1] for i in range(0, len(xs) - 1, 2)] + \
             ([xs[-1]] if len(xs) % 2 else [])
    return xs[0]


def _rsqrt(q):
    i = plsc.bitcast(q, jnp.int32)
    y = plsc.bitcast(jnp.int32(0x5F3759DF) - (i >> 1), jnp.float32)
    for _ in range(3):
        y = y * (1.5 - 0.5 * q * y * y)
    return y


def _perm_ids(e):
    e = e.astype(jnp.int32)
    return ((e >> 10) << 10) | ((e & 127) << 3) | ((e >> 7) & 7)


def _relayout_body(src, dst):
    x = src[...]
    for s in range(EBLK // 1024):
        for j in range(8):
            dst[128 * s:128 * (s + 1), 16 * j:16 * (j + 1)] = \
                x[:, 1024 * s + 128 * j:1024 * s + 128 * (j + 1)].T


def _stage_wa(w_hbm, a_hbm, w_v, a_v):
    pltpu.sync_copy(w_hbm, w_v)
    pltpu.sync_copy(a_hbm, a_v)
    arow = a_v[0, :]
    wrows = [w_v[i, :] for i in range(8)]
    return arow, wrows


def _body_a(nb_hbm, tid_hbm, et_hbm, w_hbm, a_hbm, y_hbm,
            idx_v, nbr0_v, nbr1_v, tid_v, trow_v, w_v, a_v, y_v,
            sem0, sem1):
    cid = lax.axis_index("c")
    sid = lax.axis_index("s")
    wid = sid * NC + cid

    arow, wrows = _stage_wa(w_hbm, a_hbm, w_v, a_v)
    v1 = jnp.zeros((16,), jnp.float32)
    v2 = jnp.zeros((16,), jnp.float32)
    for i in range(8):
        v1 = v1 + wrows[i] * arow[i]
        v2 = v2 + wrows[i] * arow[8 + i]
    v1s = [v1[d] for d in range(16)]
    v2s = [v2[d] for d in range(16)]

    iota = lax.iota(jnp.int32, 16)
    cds = [jnp.full((16,), d, jnp.int32) for d in range(16)]

    pltpu.sync_copy(nb_hbm.at[wid], idx_v)
    pltpu.sync_copy(tid_hbm.at[wid], tid_v)
    pltpu.async_copy(et_hbm.at[tid_v], trow_v, sem0).wait()

    def fire(c, buf, sem):
        pltpu.async_copy(et_hbm.at[idx_v.at[pl.ds(c * (C * K), C * K)]],
                         buf, sem)

    def drain(buf, sem):
        pltpu.make_async_copy(et_hbm.at[pl.ds(0, C * K)], buf, sem).wait()

    fire(0, nbr0_v, sem0)

    def super_body(ss, carry):
        c0 = 2 * ss

        def make_group_body(c, nbr_v):
            def group_body(g, carry2):
                row16 = c * C + g * 16 + iota

                tcols = [plsc.load_gather(trow_v, [row16, cds[d]])
                         for d in range(16)]
                tq = _tree_sum([c2 * c2 for c2 in tcols])
                tp = _tree_sum([tcols[d] * v1s[d] for d in range(16)])
                st = tp * jnp.minimum(1.0, _rsqrt(tq))

                row50 = (g * 16 + iota) * K
                zero = jnp.zeros((16,), jnp.float32)

                @plsc.parallel_loop(0, K, unroll=2, carry=(zero, (zero,) * 16))
                def k_loop(k, kc):
                    ssum, acc = kc
                    r = row50 + k
                    cols = [plsc.load_gather(nbr_v, [r, cds[d]])
                            for d in range(16)]
                    q = _tree_sum([c2 * c2 for c2 in cols])
                    p = _tree_sum([cols[d] * v2s[d] for d in range(16)])
                    scl = jnp.minimum(1.0, _rsqrt(q))
                    e = st + p * scl
                    e = jnp.where(e >= 0.0, e, e * 0.2)
                    w = jnp.exp(e)
                    t = w * scl
                    acc = tuple(acc[d] + t * cols[d] for d in range(16))
                    return (ssum + w, acc)

                ssum, acc = k_loop

                inv = 1.0 / ssum
                for d in range(16):
                    plsc.store_scatter(y_v, [row16, cds[d]], acc[d] * inv)
                return 0
            return group_body

        fire(c0 + 1, nbr1_v, sem1)
        drain(nbr0_v, sem0)
        lax.fori_loop(0, NG, make_group_body(c0, nbr0_v), 0)

        @pl.when(ss < NCHUNK // 2 - 1)
        def _():
            fire(c0 + 2, nbr0_v, sem0)

        drain(nbr1_v, sem1)
        lax.fori_loop(0, NG, make_group_body(c0 + 1, nbr1_v), 0)
        return 0

    lax.fori_loop(0, NCHUNK // 2, super_body, 0)
    pltpu.sync_copy(y_v, y_hbm.at[pl.ds(wid * RPW, RPW)])


def _body_b(y_hbm, uid_hbm, ut_hbm, w_hbm, a_hbm, out_hbm,
            uid_v, urow_v, y_v, w_v, a_v, out_v, sem):
    cid = lax.axis_index("c")
    sid = lax.axis_index("s")
    wid = sid * NC + cid

    _, wrows = _stage_wa(w_hbm, a_hbm, w_v, a_v)
    iota = lax.iota(jnp.int32, 16)
    cds = [jnp.full((16,), d, jnp.int32) for d in range(16)]

    pltpu.sync_copy(uid_hbm.at[wid], uid_v)
    pltpu.sync_copy(y_hbm.at[pl.ds(wid * RPW, RPW)], y_v)
    pltpu.async_copy(ut_hbm.at[uid_v], urow_v, sem).wait()

    def group_body(g, carry):
        row16 = g * 16 + iota
        ucols = [plsc.load_gather(urow_v, [row16, cds[d]])
                 for d in range(16)]
        uq = _tree_sum([c2 * c2 for c2 in ucols])
        us = jnp.minimum(1.0, _rsqrt(uq))
        gs = [ucols[i] + ucols[8 + i] for i in range(8)]
        uv = jnp.zeros((16,), jnp.float32)
        for j in range(16):
            wv = gs[0] * wrows[0][j]
            for i in range(1, 8):
                wv = wv + gs[i] * wrows[i][j]
            yj = plsc.load_gather(y_v, [row16, cds[j]])
            uv = uv + yj * wv
        uv = uv * us
        logit = 1.0 / (1.0 + jnp.exp(-uv))
        out_v[pl.ds(g * 16, 16)] = logit
        return 0

    lax.fori_loop(0, RPW // 16, group_body, 0)
    pltpu.sync_copy(out_v, out_hbm.at[pl.ds(wid * RPW, RPW)])


def kernel(u, target_ids, neighbor_ids, entity_table, user_table, W, a):
    relayout = pl.pallas_call(
        _relayout_body,
        grid=(NBLK,),
        in_specs=[pl.BlockSpec((16, EBLK), lambda i: (0, i))],
        out_specs=pl.BlockSpec((EBLK // 8, 128), lambda i: (i, 0)),
        out_shape=jax.ShapeDtypeStruct((ZROWS, 128), jnp.float32),
    )
    et16 = relayout(entity_table.T).reshape(ZVIEW, DIM)
    ut16 = relayout(user_table.T).reshape(ZVIEW, DIM)

    nb2d = _perm_ids(neighbor_ids).reshape(NW, RPW * K)
    tids = _perm_ids(target_ids).reshape(NW, RPW)
    uids = _perm_ids(u).reshape(NW, RPW)
    mesh = plsc.VectorSubcoreMesh(core_axis_name="c", subcore_axis_name="s")
    fn_a = pl.kernel(
        _body_a,
        out_type=jax.ShapeDtypeStruct((B, DIM), jnp.float32),
        mesh=mesh,
        scratch_types=[
            pltpu.VMEM((RPW * K,), jnp.int32),
            pltpu.VMEM((C * K, DIM), jnp.float32),
            pltpu.VMEM((C * K, DIM), jnp.float32),
            pltpu.VMEM((RPW,), jnp.int32),
            pltpu.VMEM((RPW, DIM), jnp.float32),
            pltpu.VMEM((8, DIM), jnp.float32),
            pltpu.VMEM((1, DIM), jnp.float32),
            pltpu.VMEM((RPW, DIM), jnp.float32),
            pltpu.SemaphoreType.DMA,
            pltpu.SemaphoreType.DMA,
        ],
        **_PARAMS,
    )
    y = fn_a(nb2d, tids, et16, W, a)
    fn_b = pl.kernel(
        _body_b,
        out_type=jax.ShapeDtypeStruct((B,), jnp.float32),
        mesh=mesh,
        scratch_types=[
            pltpu.VMEM((RPW,), jnp.int32),
            pltpu.VMEM((RPW, DIM), jnp.float32),
            pltpu.VMEM((RPW, DIM), jnp.float32),
            pltpu.VMEM((8, DIM), jnp.float32),
            pltpu.VMEM((1, DIM), jnp.float32),
            pltpu.VMEM((RPW,), jnp.float32),
            pltpu.SemaphoreType.DMA,
        ],
        **_PARAMS,
    )
    return fn_b(y, uids, ut16, W, a)
